# baseline (speedup 1.0000x reference)
import functools

import jax
import jax.numpy as jnp
from jax import lax
from jax.experimental import pallas as pl
from jax.experimental.pallas import tpu as pltpu
from jax.experimental.pallas import tpu_sc as plsc

B = 16384
NU = 1000000
NI = 100000
F = 32
R = 16
NC = 2
NS = 16
NW = NC * NS
BPW = B // NW

BB = BPW
NB = B // BB


def _sc_gather_body(uid_hbm, iid_hbm, wut_hbm, wit_hbm, at_hbm,
                    ub_hbm, ib_hbm,
                    ue_out, wi_out, a_out, ub_out, ib_out,
                    uidx_v, iidx_v, ue_v, wiv_v, a_v, ub_v, ib_v, sem):
    wid = lax.axis_index("s") * NC + lax.axis_index("c")
    base = wid * BPW
    pltpu.sync_copy(uid_hbm.at[pl.ds(base, BPW)], uidx_v)
    pltpu.sync_copy(iid_hbm.at[pl.ds(base, BPW)], iidx_v)
    copies = []
    for f in range(F):
        copies.append(pltpu.async_copy(
            wut_hbm.at[pl.ds(f * NU, NU)].at[uidx_v],
            ue_v.at[pl.ds(f * BPW, BPW)], sem))
        copies.append(pltpu.async_copy(
            wit_hbm.at[pl.ds(f * NI, NI)].at[iidx_v],
            wiv_v.at[pl.ds(f * BPW, BPW)], sem))
    for r in range(R):
        copies.append(pltpu.async_copy(
            at_hbm.at[pl.ds(r * NI, NI)].at[iidx_v],
            a_v.at[pl.ds(r * BPW, BPW)], sem))
    copies.append(pltpu.async_copy(ub_hbm.at[uidx_v], ub_v, sem))
    copies.append(pltpu.async_copy(ib_hbm.at[iidx_v], ib_v, sem))
    for c in copies:
        c.wait()
    pltpu.sync_copy(ue_v, ue_out.at[pl.ds(base * F, BPW * F)])
    pltpu.sync_copy(wiv_v, wi_out.at[pl.ds(base * F, BPW * F)])
    pltpu.sync_copy(a_v, a_out.at[pl.ds(base * R, BPW * R)])
    pltpu.sync_copy(ub_v, ub_out.at[pl.ds(base, BPW)])
    pltpu.sync_copy(ib_v, ib_out.at[pl.ds(base, BPW)])


@functools.cache
def _make_sc_gather():
    return pl.kernel(
        _sc_gather_body,
        out_type=(
            jax.ShapeDtypeStruct((B * F,), jnp.float32),
            jax.ShapeDtypeStruct((B * F,), jnp.float32),
            jax.ShapeDtypeStruct((B * R,), jnp.float32),
            jax.ShapeDtypeStruct((B,), jnp.float32),
            jax.ShapeDtypeStruct((B,), jnp.float32),
        ),
        mesh=plsc.VectorSubcoreMesh(core_axis_name="c", subcore_axis_name="s",
                                    num_cores=NC, num_subcores=NS),
        scratch_types=[
            pltpu.VMEM((BPW,), jnp.int32),
            pltpu.VMEM((BPW,), jnp.int32),
            pltpu.VMEM((BPW * F,), jnp.float32),
            pltpu.VMEM((BPW * F,), jnp.float32),
            pltpu.VMEM((BPW * R,), jnp.float32),
            pltpu.VMEM((BPW,), jnp.float32),
            pltpu.VMEM((BPW,), jnp.float32),
            pltpu.SemaphoreType.DMA,
        ],
    )


def _tc_body(uft_ref, ift_ref, ue_ref, wiv_ref, a_ref, ub_ref, ib_ref,
             wuf_ref, wif_ref, bmat_ref, ufb_ref, ifb_ref, off_ref, out_ref):
    uft = uft_ref[...]
    ift = ift_ref[...]
    ue = ue_ref[0]
    wiv = wiv_ref[0]
    a = a_ref[0]
    wuf = wuf_ref[...]
    wif = wif_ref[...]
    bmat = bmat_ref[...]

    dn = (((0,), (0,)), ((), ()))
    ie = wiv + lax.dot_general(bmat, a, dn,
                               preferred_element_type=jnp.float32)
    d = (lax.dot_general(wuf, uft, dn, preferred_element_type=jnp.float32)
         + lax.dot_general(wif, ift, dn, preferred_element_type=jnp.float32))
    s = ue + ie + d

    wuf2 = jnp.sum(wuf * wuf, axis=1, keepdims=True)
    wif2 = jnp.sum(wif * wif, axis=1, keepdims=True)
    sq = (jnp.sum(ue * ue, axis=0, keepdims=True)
          + jnp.sum(ie * ie, axis=0, keepdims=True)
          + jnp.sum(uft * uft * wuf2, axis=0, keepdims=True)
          + jnp.sum(ift * ift * wif2, axis=0, keepdims=True))
    quad = jnp.sum(s * s, axis=0, keepdims=True) - sq

    fb = (jnp.sum(uft * ufb_ref[...], axis=0, keepdims=True)
          + jnp.sum(ift * ifb_ref[...], axis=0, keepdims=True))
    out_ref[...] = (0.5 * quad + ub_ref[...] + ib_ref[...] + fb
                    + off_ref[0, 0])


def kernel(user_ids, item_ids, user_feats, item_feats, W_user, W_item,
           W_ufeat, W_ifeat, user_bias, item_bias, user_feat_bias,
           item_feat_bias, offset, A, Bmat):
    uid = user_ids.astype(jnp.int32)
    iid = item_ids.astype(jnp.int32)
    wut = W_user.T.reshape(NU * F)
    wit = W_item.T.reshape(NI * F)
    at = A.T.reshape(NI * R)
    ue_f, wi_f, a_f, ub, ib = _make_sc_gather()(uid, iid, wut, wit, at,
                                                user_bias, item_bias)
    ue3 = ue_f.reshape(NW, F, BPW)
    wi3 = wi_f.reshape(NW, F, BPW)
    a3 = a_f.reshape(NW, R, BPW)

    nuf = user_feats.shape[1]
    nif = item_feats.shape[1]
    out = pl.pallas_call(
        _tc_body,
        grid=(NB,),
        in_specs=[
            pl.BlockSpec((nuf, BB), lambda i: (0, i)),
            pl.BlockSpec((nif, BB), lambda i: (0, i)),
            pl.BlockSpec((1, F, BB), lambda i: (i, 0, 0)),
            pl.BlockSpec((1, F, BB), lambda i: (i, 0, 0)),
            pl.BlockSpec((1, R, BB), lambda i: (i, 0, 0)),
            pl.BlockSpec((1, BB), lambda i: (0, i)),
            pl.BlockSpec((1, BB), lambda i: (0, i)),
            pl.BlockSpec((nuf, F), lambda i: (0, 0)),
            pl.BlockSpec((nif, F), lambda i: (0, 0)),
            pl.BlockSpec((R, F), lambda i: (0, 0)),
            pl.BlockSpec((nuf, 1), lambda i: (0, 0)),
            pl.BlockSpec((nif, 1), lambda i: (0, 0)),
            pl.BlockSpec((1, 1), lambda i: (0, 0)),
        ],
        out_specs=pl.BlockSpec((1, BB), lambda i: (0, i)),
        out_shape=jax.ShapeDtypeStruct((1, B), jnp.float32),
    )(user_feats.T, item_feats.T, ue3, wi3, a3,
      ub.reshape(1, B), ib.reshape(1, B),
      W_ufeat, W_ifeat, Bmat,
      user_feat_bias.reshape(nuf, 1), item_feat_bias.reshape(nif, 1),
      offset.reshape(1, 1))
    return out.reshape(B)


# speedup vs baseline: 4.4501x; 4.4501x over previous
import functools

import jax
import jax.numpy as jnp
from jax import lax
from jax.experimental import pallas as pl
from jax.experimental.pallas import tpu as pltpu
from jax.experimental.pallas import tpu_sc as plsc

B = 16384
NU = 1000000
NI = 100000
F = 32
R = 16
NC = 2
NS = 16
NW = NC * NS
BPW = B // NW

BB = BPW
NB = B // BB


def _sc_gather_body(uid_hbm, iid_hbm, wut_hbm, wit_hbm, at_hbm,
                    ub_hbm, ib_hbm,
                    ue_out, wi_out, a_out, ub_out, ib_out,
                    uidx_v, iidx_v, ub2_v, ue_v, wiv_v, a_v, ub_v, ib_v, sem):
    wid = lax.axis_index("s") * NC + lax.axis_index("c")
    base = wid * BPW
    pltpu.sync_copy(uid_hbm.at[pl.ds(base, BPW)], uidx_v)
    pltpu.sync_copy(iid_hbm.at[pl.ds(base, BPW)], iidx_v)
    for c in range(BPW // 16):
        sl = pl.ds(c * 16, 16)
        u32 = uidx_v[sl] * F
        for r in range(8):
            ub2_v[pl.ds(r * BPW + c * 16, 16)] = u32 + r
    copies = []
    for f in range(F):
        copies.append(pltpu.async_copy(
            wut_hbm.at[pl.ds(f & ~7, F * NU - (f & ~7))]
                   .at[ub2_v.at[pl.ds((f & 7) * BPW, BPW)]],
            ue_v.at[pl.ds(f * BPW, BPW)], sem))
        copies.append(pltpu.async_copy(
            wit_hbm.at[pl.ds(f * NI, NI)].at[iidx_v],
            wiv_v.at[pl.ds(f * BPW, BPW)], sem))
    for r in range(R):
        copies.append(pltpu.async_copy(
            at_hbm.at[pl.ds(r * NI, NI)].at[iidx_v],
            a_v.at[pl.ds(r * BPW, BPW)], sem))
    copies.append(pltpu.async_copy(ub_hbm.at[uidx_v], ub_v, sem))
    copies.append(pltpu.async_copy(ib_hbm.at[iidx_v], ib_v, sem))
    for c in copies:
        c.wait()
    pltpu.sync_copy(ue_v, ue_out.at[pl.ds(base * F, BPW * F)])
    pltpu.sync_copy(wiv_v, wi_out.at[pl.ds(base * F, BPW * F)])
    pltpu.sync_copy(a_v, a_out.at[pl.ds(base * R, BPW * R)])
    pltpu.sync_copy(ub_v, ub_out.at[pl.ds(base, BPW)])
    pltpu.sync_copy(ib_v, ib_out.at[pl.ds(base, BPW)])


@functools.cache
def _make_sc_gather():
    return pl.kernel(
        _sc_gather_body,
        out_type=(
            jax.ShapeDtypeStruct((B * F,), jnp.float32),
            jax.ShapeDtypeStruct((B * F,), jnp.float32),
            jax.ShapeDtypeStruct((B * R,), jnp.float32),
            jax.ShapeDtypeStruct((B,), jnp.float32),
            jax.ShapeDtypeStruct((B,), jnp.float32),
        ),
        mesh=plsc.VectorSubcoreMesh(core_axis_name="c", subcore_axis_name="s",
                                    num_cores=NC, num_subcores=NS),
        scratch_types=[
            pltpu.VMEM((BPW,), jnp.int32),
            pltpu.VMEM((BPW,), jnp.int32),
            pltpu.VMEM((8 * BPW,), jnp.int32),
            pltpu.VMEM((BPW * F,), jnp.float32),
            pltpu.VMEM((BPW * F,), jnp.float32),
            pltpu.VMEM((BPW * R,), jnp.float32),
            pltpu.VMEM((BPW,), jnp.float32),
            pltpu.VMEM((BPW,), jnp.float32),
            pltpu.SemaphoreType.DMA,
        ],
    )


def _tc_body(uft_ref, ift_ref, ue_ref, wiv_ref, a_ref, ub_ref, ib_ref,
             wuf_ref, wif_ref, bmat_ref, ufb_ref, ifb_ref, off_ref, out_ref):
    uft = uft_ref[...]
    ift = ift_ref[...]
    ue = ue_ref[0]
    wiv = wiv_ref[0]
    a = a_ref[0]
    wuf = wuf_ref[...]
    wif = wif_ref[...]
    bmat = bmat_ref[...]

    dn = (((0,), (0,)), ((), ()))
    ie = wiv + lax.dot_general(bmat, a, dn,
                               preferred_element_type=jnp.float32)
    d = (lax.dot_general(wuf, uft, dn, preferred_element_type=jnp.float32)
         + lax.dot_general(wif, ift, dn, preferred_element_type=jnp.float32))
    s = ue + ie + d

    wuf2 = jnp.sum(wuf * wuf, axis=1, keepdims=True)
    wif2 = jnp.sum(wif * wif, axis=1, keepdims=True)
    sq = (jnp.sum(ue * ue, axis=0, keepdims=True)
          + jnp.sum(ie * ie, axis=0, keepdims=True)
          + jnp.sum(uft * uft * wuf2, axis=0, keepdims=True)
          + jnp.sum(ift * ift * wif2, axis=0, keepdims=True))
    quad = jnp.sum(s * s, axis=0, keepdims=True) - sq

    fb = (jnp.sum(uft * ufb_ref[...], axis=0, keepdims=True)
          + jnp.sum(ift * ifb_ref[...], axis=0, keepdims=True))
    out_ref[...] = (0.5 * quad + ub_ref[...] + ib_ref[...] + fb
                    + off_ref[0, 0])


def kernel(user_ids, item_ids, user_feats, item_feats, W_user, W_item,
           W_ufeat, W_ifeat, user_bias, item_bias, user_feat_bias,
           item_feat_bias, offset, A, Bmat):
    uid = user_ids.astype(jnp.int32)
    iid = item_ids.astype(jnp.int32)
    wu128 = lax.optimization_barrier(W_user.reshape(NU * F // 128, 128))
    wut = wu128.reshape(NU * F)
    wit = W_item.T.reshape(NI * F)
    at = A.T.reshape(NI * R)
    ue_f, wi_f, a_f, ub, ib = _make_sc_gather()(uid, iid, wut, wit, at,
                                                user_bias, item_bias)
    ue3 = ue_f.reshape(NW, F, BPW)
    wi3 = wi_f.reshape(NW, F, BPW)
    a3 = a_f.reshape(NW, R, BPW)

    nuf = user_feats.shape[1]
    nif = item_feats.shape[1]
    out = pl.pallas_call(
        _tc_body,
        grid=(NB,),
        in_specs=[
            pl.BlockSpec((nuf, BB), lambda i: (0, i)),
            pl.BlockSpec((nif, BB), lambda i: (0, i)),
            pl.BlockSpec((1, F, BB), lambda i: (i, 0, 0)),
            pl.BlockSpec((1, F, BB), lambda i: (i, 0, 0)),
            pl.BlockSpec((1, R, BB), lambda i: (i, 0, 0)),
            pl.BlockSpec((1, BB), lambda i: (0, i)),
            pl.BlockSpec((1, BB), lambda i: (0, i)),
            pl.BlockSpec((nuf, F), lambda i: (0, 0)),
            pl.BlockSpec((nif, F), lambda i: (0, 0)),
            pl.BlockSpec((R, F), lambda i: (0, 0)),
            pl.BlockSpec((nuf, 1), lambda i: (0, 0)),
            pl.BlockSpec((nif, 1), lambda i: (0, 0)),
            pl.BlockSpec((1, 1), lambda i: (0, 0)),
        ],
        out_specs=pl.BlockSpec((1, BB), lambda i: (0, i)),
        out_shape=jax.ShapeDtypeStruct((1, B), jnp.float32),
    )(user_feats.T, item_feats.T, ue3, wi3, a3,
      ub.reshape(1, B), ib.reshape(1, B),
      W_ufeat, W_ifeat, Bmat,
      user_feat_bias.reshape(nuf, 1), item_feat_bias.reshape(nif, 1),
      offset.reshape(1, 1))
    return out.reshape(B)


# speedup vs baseline: 4.5104x; 1.0136x over previous
import functools

import jax
import jax.numpy as jnp
from jax import lax
from jax.experimental import pallas as pl
from jax.experimental.pallas import tpu as pltpu
from jax.experimental.pallas import tpu_sc as plsc

B = 16384
NU = 1000000
NI = 100000
F = 32
R = 16
NC = 2
NS = 16
NW = NC * NS
BPW = B // NW

BB = BPW
NB = B // BB


def _sc_gather_body(uid_hbm, iid_hbm, wut_hbm, wit_hbm, at_hbm,
                    ub_hbm, ib_hbm,
                    ue_out, wi_out, a_out, ub_out, ib_out,
                    uidx_v, iidx_v, ub2_v, ue_v, wiv_v, a_v, ub_v, ib_v, sem):
    wid = lax.axis_index("s") * NC + lax.axis_index("c")
    base = wid * BPW
    pltpu.sync_copy(uid_hbm.at[pl.ds(base, BPW)], uidx_v)
    pltpu.sync_copy(iid_hbm.at[pl.ds(base, BPW)], iidx_v)
    copies = [pltpu.async_copy(wut_hbm.at[uidx_v], ub2_v, sem)]
    for f in range(F):
        copies.append(pltpu.async_copy(
            wit_hbm.at[pl.ds(f * NI, NI)].at[iidx_v],
            wiv_v.at[pl.ds(f * BPW, BPW)], sem))
    for r in range(R):
        copies.append(pltpu.async_copy(
            at_hbm.at[pl.ds(r * NI, NI)].at[iidx_v],
            a_v.at[pl.ds(r * BPW, BPW)], sem))
    copies.append(pltpu.async_copy(ub_hbm.at[uidx_v], ub_v, sem))
    copies.append(pltpu.async_copy(ib_hbm.at[iidx_v], ib_v, sem))
    for c in copies:
        c.wait()
    iota16 = lax.iota(jnp.int32, 16)

    cols = [jnp.full((16,), f, jnp.int32) for f in range(F)]

    def _transpose_c(c, carry):
        rows = iota16 + c * 16
        for f in range(F):
            ue_v[pl.ds(f * BPW + c * 16, 16)] = plsc.load_gather(
                ub2_v, [rows, cols[f]])
        return carry

    lax.fori_loop(0, BPW // 16, _transpose_c, 0)
    pltpu.sync_copy(ue_v, ue_out.at[pl.ds(base * F, BPW * F)])
    pltpu.sync_copy(wiv_v, wi_out.at[pl.ds(base * F, BPW * F)])
    pltpu.sync_copy(a_v, a_out.at[pl.ds(base * R, BPW * R)])
    pltpu.sync_copy(ub_v, ub_out.at[pl.ds(base, BPW)])
    pltpu.sync_copy(ib_v, ib_out.at[pl.ds(base, BPW)])


@functools.cache
def _make_sc_gather():
    return pl.kernel(
        _sc_gather_body,
        out_type=(
            jax.ShapeDtypeStruct((B * F,), jnp.float32),
            jax.ShapeDtypeStruct((B * F,), jnp.float32),
            jax.ShapeDtypeStruct((B * R,), jnp.float32),
            jax.ShapeDtypeStruct((B,), jnp.float32),
            jax.ShapeDtypeStruct((B,), jnp.float32),
        ),
        mesh=plsc.VectorSubcoreMesh(core_axis_name="c", subcore_axis_name="s",
                                    num_cores=NC, num_subcores=NS),
        compiler_params=pltpu.CompilerParams(needs_layout_passes=False,
                                             use_tc_tiling_on_sc=False),
        scratch_types=[
            pltpu.VMEM((BPW,), jnp.int32),
            pltpu.VMEM((BPW,), jnp.int32),
            pltpu.VMEM((BPW, F), jnp.float32),
            pltpu.VMEM((BPW * F,), jnp.float32),
            pltpu.VMEM((BPW * F,), jnp.float32),
            pltpu.VMEM((BPW * R,), jnp.float32),
            pltpu.VMEM((BPW,), jnp.float32),
            pltpu.VMEM((BPW,), jnp.float32),
            pltpu.SemaphoreType.DMA,
        ],
    )


def _tc_body(uft_ref, ift_ref, ue_ref, wiv_ref, a_ref, ub_ref, ib_ref,
             wuf_ref, wif_ref, bmat_ref, ufb_ref, ifb_ref, off_ref, out_ref):
    uft = uft_ref[...]
    ift = ift_ref[...]
    ue = ue_ref[0]
    wiv = wiv_ref[0]
    a = a_ref[0]
    wuf = wuf_ref[...]
    wif = wif_ref[...]
    bmat = bmat_ref[...]

    dn = (((0,), (0,)), ((), ()))
    ie = wiv + lax.dot_general(bmat, a, dn,
                               preferred_element_type=jnp.float32)
    d = (lax.dot_general(wuf, uft, dn, preferred_element_type=jnp.float32)
         + lax.dot_general(wif, ift, dn, preferred_element_type=jnp.float32))
    s = ue + ie + d

    wuf2 = jnp.sum(wuf * wuf, axis=1, keepdims=True)
    wif2 = jnp.sum(wif * wif, axis=1, keepdims=True)
    sq = (jnp.sum(ue * ue, axis=0, keepdims=True)
          + jnp.sum(ie * ie, axis=0, keepdims=True)
          + jnp.sum(uft * uft * wuf2, axis=0, keepdims=True)
          + jnp.sum(ift * ift * wif2, axis=0, keepdims=True))
    quad = jnp.sum(s * s, axis=0, keepdims=True) - sq

    fb = (jnp.sum(uft * ufb_ref[...], axis=0, keepdims=True)
          + jnp.sum(ift * ifb_ref[...], axis=0, keepdims=True))
    out_ref[...] = (0.5 * quad + ub_ref[...] + ib_ref[...] + fb
                    + off_ref[0, 0])


def kernel(user_ids, item_ids, user_feats, item_feats, W_user, W_item,
           W_ufeat, W_ifeat, user_bias, item_bias, user_feat_bias,
           item_feat_bias, offset, A, Bmat):
    uid = user_ids.astype(jnp.int32)
    iid = item_ids.astype(jnp.int32)
    wu128 = lax.optimization_barrier(W_user.reshape(NU * F // 128, 128))
    wut = wu128.reshape(NU, F)
    wit = W_item.T.reshape(NI * F)
    at = A.T.reshape(NI * R)
    ue_f, wi_f, a_f, ub, ib = _make_sc_gather()(uid, iid, wut, wit, at,
                                                user_bias, item_bias)
    ue3 = ue_f.reshape(NW, F, BPW)
    wi3 = wi_f.reshape(NW, F, BPW)
    a3 = a_f.reshape(NW, R, BPW)

    nuf = user_feats.shape[1]
    nif = item_feats.shape[1]
    out = pl.pallas_call(
        _tc_body,
        grid=(NB,),
        in_specs=[
            pl.BlockSpec((nuf, BB), lambda i: (0, i)),
            pl.BlockSpec((nif, BB), lambda i: (0, i)),
            pl.BlockSpec((1, F, BB), lambda i: (i, 0, 0)),
            pl.BlockSpec((1, F, BB), lambda i: (i, 0, 0)),
            pl.BlockSpec((1, R, BB), lambda i: (i, 0, 0)),
            pl.BlockSpec((1, BB), lambda i: (0, i)),
            pl.BlockSpec((1, BB), lambda i: (0, i)),
            pl.BlockSpec((nuf, F), lambda i: (0, 0)),
            pl.BlockSpec((nif, F), lambda i: (0, 0)),
            pl.BlockSpec((R, F), lambda i: (0, 0)),
            pl.BlockSpec((nuf, 1), lambda i: (0, 0)),
            pl.BlockSpec((nif, 1), lambda i: (0, 0)),
            pl.BlockSpec((1, 1), lambda i: (0, 0)),
        ],
        out_specs=pl.BlockSpec((1, BB), lambda i: (0, i)),
        out_shape=jax.ShapeDtypeStruct((1, B), jnp.float32),
    )(user_feats.T, item_feats.T, ue3, wi3, a3,
      ub.reshape(1, B), ib.reshape(1, B),
      W_ufeat, W_ifeat, Bmat,
      user_feat_bias.reshape(nuf, 1), item_feat_bias.reshape(nif, 1),
      offset.reshape(1, 1))
    return out.reshape(B)
